# Initial kernel scaffold; baseline (speedup 1.0000x reference)
import jax
import jax.numpy as jnp
from jax.experimental import pallas as pl


def kernel(patches, kernel_2d, inds_t, inds_y, inds_x):
    raise NotImplementedError("write your pallas kernel here")


# speedup vs baseline: 219.4490x; 219.4490x over previous
import functools

import jax
import jax.numpy as jnp
from jax import lax
from jax.experimental import pallas as pl
from jax.experimental.pallas import tpu as pltpu
from jax.experimental.pallas import tpu_sc as plsc

PS = 8
T, C, H, W = 8, 3, 480, 480
N = 100000
CP2 = C * PS * PS
HW = H * W
CHW = C * HW
TCHW = T * CHW
NPAD = 100352
CH1 = 6272
CH2 = 3136
SLC = CHW // 16
SUB = 64
NG2 = CH2 // 16
NR = 16 * CP2 // 128

_i32 = jnp.int32
_f32 = jnp.float32


def _lane_consts():
    iot = lax.iota(_i32, 16)
    pat = (iot & 7) + W * (iot >> 3)
    offs = []
    for k in range(12):
        q = 16 * k
        offs.append(pat + (q // 64) * HW + ((q % 64) // 8) * W)
    return iot, offs


def _bcast_lane(v, j, iot):
    s = jnp.sum(jnp.where(iot == j, v, _i32(0)))
    return jnp.broadcast_to(s, (16,))


def _scatter_body(indt, indy, indx, pat2, vid, tch, ych, xch, posb, sb,
                  pidb, valb, idxb, zbuf, wbuf, spm, semv, semsc):
    cid = lax.axis_index("c")
    sid = lax.axis_index("s")
    base = sid * CH1
    pltpu.sync_copy(indt.at[pl.ds(base, CH1)], tch)
    pltpu.sync_copy(indy.at[pl.ds(base, CH1)], ych)
    pltpu.sync_copy(indx.at[pl.ds(base, CH1)], xch)
    z16 = jnp.zeros((16,), _f32)

    def _zb(i, carry):
        zbuf[pl.ds(i * 16, 16)] = z16
        return carry

    lax.fori_loop(0, 512, _zb, 0)
    iot, offs = _lane_consts()

    zslices = [(o * 8192, 8192) for o in range(5)] + [(40960, 2240)]
    for f_loc in range(4):
        f = cid * 4 + f_loc
        for (o, sz) in zslices:
            pltpu.sync_copy(zbuf.at[pl.ds(0, sz)],
                            spm.at[pl.ds(sid * SLC + o, sz)])
        plsc.subcore_barrier()

        def _compact(g, cnt):
            t = tch[pl.ds(g * 16, 16)]
            m = t == f
            pos = g * 16 + iot
            mi = jnp.where(m, _i32(1), _i32(0))
            dst = cnt + plsc.cumsum(mi) - 1
            plsc.store_scatter(posb, [dst], pos, mask=m)
            return cnt + jnp.sum(mi)

        cnt = lax.fori_loop(0, CH1 // 16, _compact, _i32(0))
        nsub = (cnt + (SUB - 1)) >> 6

        def _sub(j, carry):
            off = j * SUB
            for q in range(4):
                lv = (off + q * 16 + iot) < cnt
                pos = posb[pl.ds(off + q * 16, 16)]
                pos = jnp.where(lv, pos, _i32(0))
                yv = plsc.load_gather(ych, [pos])
                xv = plsc.load_gather(xch, [pos])
                sb[pl.ds(q * 16, 16)] = yv * W + xv
                pidb[pl.ds(q * 16, 16)] = pos + base
            pltpu.async_copy(pat2.at[pidb], valb, semv).wait()
            rem = jnp.minimum(cnt - off, SUB)

            def _pidx(p, carry2):
                sv16 = sb[pl.ds((p >> 4) * 16, 16)]
                sbc = _bcast_lane(sv16, p & 15, iot)
                for k in range(12):
                    idxb[p, pl.ds(k * 16, 16)] = sbc + offs[k]
                pltpu.async_copy(valb.at[p], spm.at[idxb.at[p]],
                                 semsc, add=True)

                @pl.when(p >= 1)
                def _():
                    pltpu.make_async_copy(valb.at[p - 1],
                                          spm.at[idxb.at[p - 1]],
                                          semsc).wait()

                return carry2

            lax.fori_loop(0, rem, _pidx, 0)

            @pl.when(rem >= 1)
            def _():
                pltpu.make_async_copy(valb.at[rem - 1],
                                      spm.at[idxb.at[rem - 1]],
                                      semsc).wait()

            return carry

        lax.fori_loop(0, nsub, _sub, 0)
        plsc.subcore_barrier()
        wslices = [(o * 8192, 8192) for o in range(5)] + [(40960, 2240)]
        for (o, sz) in wslices:
            pltpu.sync_copy(spm.at[pl.ds(sid * SLC + o, sz)],
                            wbuf.at[pl.ds(0, sz)])
            pltpu.sync_copy(wbuf.at[pl.ds(0, sz)],
                            vid.at[pl.ds(f * CHW + sid * SLC + o, sz)])
        plsc.subcore_barrier()


def _make_scatter():
    mesh = plsc.VectorSubcoreMesh(core_axis_name="c", subcore_axis_name="s")
    return pl.kernel(
        _scatter_body,
        out_type=jax.ShapeDtypeStruct((TCHW,), _f32),
        mesh=mesh,
        compiler_params=pltpu.CompilerParams(needs_layout_passes=False,
                                             use_tc_tiling_on_sc=False),
        scratch_types=[
            pltpu.VMEM((CH1,), _i32),
            pltpu.VMEM((CH1,), _i32),
            pltpu.VMEM((CH1,), _i32),
            pltpu.VMEM((CH1,), _i32),
            pltpu.VMEM((SUB,), _i32),
            pltpu.VMEM((SUB,), _i32),
            pltpu.VMEM((SUB, CP2), _f32),
            pltpu.VMEM((SUB, CP2), _i32),
            pltpu.VMEM((8192,), _f32),
            pltpu.VMEM((8192,), _f32),
            pltpu.VMEM_SHARED((CHW,), _f32),
            pltpu.SemaphoreType.DMA,
            pltpu.SemaphoreType.DMA,
        ],
    )


def _conv_body(vref, wref, oref):
    x = vref[0]
    xl = jnp.concatenate([x[:, 1:2], x[:, :-1]], axis=1)
    xr = jnp.concatenate([x[:, 1:], x[:, -2:-1]], axis=1)
    acc = None
    for jcol, xc in ((0, xl), (1, x), (2, xr)):
        up = jnp.concatenate([xc[1:2], xc[:-1]], axis=0)
        dn = jnp.concatenate([xc[1:], xc[-2:-1]], axis=0)
        for irow, xx in ((0, up), (1, xc), (2, dn)):
            term = xx * wref[irow, jcol]
            acc = term if acc is None else acc + term
    oref[0] = acc


def _conv(vid3, w33):
    return pl.pallas_call(
        _conv_body,
        out_shape=jax.ShapeDtypeStruct((T * C, H, W), _f32),
        grid=(T * C,),
        in_specs=[
            pl.BlockSpec((1, H, W), lambda i: (i, 0, 0)),
            pl.BlockSpec((3, 3), lambda i: (0, 0)),
        ],
        out_specs=pl.BlockSpec((1, H, W), lambda i: (i, 0, 0)),
    )(vid3, w33)


def _gather_body(indt, indy, indx, vcv, out, tch, ych, xch,
                 idxa, idxb, gba, gbb, sema, semb):
    cid = lax.axis_index("c")
    sid = lax.axis_index("s")
    wid = sid * 2 + cid
    base = wid * CH2
    pltpu.sync_copy(indt.at[pl.ds(base, CH2)], tch)
    pltpu.sync_copy(indy.at[pl.ds(base, CH2)], ych)
    pltpu.sync_copy(indx.at[pl.ds(base, CH2)], xch)
    iot, offs = _lane_consts()
    obase = base * CP2 // 128

    def _group(g, ib, gb, sem):
        t = tch[pl.ds(g * 16, 16)]
        y = ych[pl.ds(g * 16, 16)]
        x = xch[pl.ds(g * 16, 16)]
        valid = t >= 0
        sv = t * CHW + y * W + x
        sv = jnp.where(valid, sv, ((g * 16 + iot) & 1023) * 192)
        for p in range(16):
            sbc = _bcast_lane(sv, p, iot)
            for k in range(12):
                flat = p * CP2 + k * 16
                ib[flat // 128, pl.ds(flat % 128, 16)] = sbc + offs[k]
        for r in range(NR):
            pltpu.async_copy(vcv.at[ib.at[r]], gb.at[r], sem)

    def _drain(g, ib, gb, sem):
        for r in range(NR):
            pltpu.make_async_copy(vcv.at[ib.at[r]], gb.at[r], sem).wait()
        pltpu.sync_copy(gb, out.at[pl.ds(obase + g * NR, NR)])

    def _pair(gg, carry):
        g0 = gg * 2
        _group(g0, idxa, gba, sema)

        @pl.when(gg >= 1)
        def _():
            _drain(g0 - 1, idxb, gbb, semb)

        _group(g0 + 1, idxb, gbb, semb)
        _drain(g0, idxa, gba, sema)
        return carry

    lax.fori_loop(0, NG2 // 2, _pair, 0)
    _drain(NG2 - 1, idxb, gbb, semb)


def _make_gather():
    mesh = plsc.VectorSubcoreMesh(core_axis_name="c", subcore_axis_name="s")
    return pl.kernel(
        _gather_body,
        out_type=jax.ShapeDtypeStruct((NPAD * CP2 // 128, 128), _f32),
        mesh=mesh,
        compiler_params=pltpu.CompilerParams(needs_layout_passes=False,
                                             use_tc_tiling_on_sc=False),
        scratch_types=[
            pltpu.VMEM((CH2,), _i32),
            pltpu.VMEM((CH2,), _i32),
            pltpu.VMEM((CH2,), _i32),
            pltpu.VMEM((NR, 128), _i32),
            pltpu.VMEM((NR, 128), _i32),
            pltpu.VMEM((NR, 128), _f32),
            pltpu.VMEM((NR, 128), _f32),
            pltpu.SemaphoreType.DMA,
            pltpu.SemaphoreType.DMA,
        ],
    )


def kernel(patches, kernel_2d, inds_t, inds_y, inds_x):
    pat2 = patches.reshape(N, CP2)
    pad = NPAD - N
    tpad = jnp.concatenate([inds_t.astype(_i32), jnp.full((pad,), -1, _i32)])
    ypad = jnp.concatenate([inds_y.astype(_i32), jnp.zeros((pad,), _i32)])
    xpad = jnp.concatenate([inds_x.astype(_i32), jnp.zeros((pad,), _i32)])
    vid = _make_scatter()(tpad, ypad, xpad, pat2)
    vcv = _conv(vid.reshape(T * C, H, W), kernel_2d.reshape(3, 3))
    y2d = _make_gather()(tpad, ypad, xpad, vcv.reshape(TCHW))
    return y2d.reshape(-1)[: N * CP2].reshape(1, N, 1, CP2)


# speedup vs baseline: 221.6364x; 1.0100x over previous
import functools

import jax
import jax.numpy as jnp
from jax import lax
from jax.experimental import pallas as pl
from jax.experimental.pallas import tpu as pltpu
from jax.experimental.pallas import tpu_sc as plsc

PS = 8
T, C, H, W = 8, 3, 480, 480
WP = 512
N = 100000
CP2 = C * PS * PS
HWP = H * WP
CHWP = C * HWP
TCHWP = T * CHWP
NPAD = 100352
CH1 = 6272
CH2 = 3136
SLC = CHWP // 16
SUB = 64
NG2 = CH2 // 16
NR = 16 * CP2 // 128
NW = 32
LASTP = (N - (NW - 1) * CH2) // 32
OROWS = N * CP2 // 128

_i32 = jnp.int32
_f32 = jnp.float32


def _lane_consts():
    iot = lax.iota(_i32, 16)
    pat = (iot & 7) + WP * (iot >> 3)
    offs = []
    for k in range(12):
        q = 16 * k
        offs.append(pat + (q // 64) * HWP + ((q % 64) // 8) * WP)
    return iot, offs


def _bcast_lane(v, j, iot):
    s = jnp.sum(jnp.where(iot == j, v, _i32(0)))
    return jnp.broadcast_to(s, (16,))


def _scatter_body(indt, indy, indx, pat2, vid, tch, ych, xch, posb, sb,
                  pidb, valb, idxb, zbuf, wbuf, spm, semv, semsc):
    cid = lax.axis_index("c")
    sid = lax.axis_index("s")
    base = sid * CH1
    pltpu.sync_copy(indt.at[pl.ds(base, CH1)], tch)
    pltpu.sync_copy(indy.at[pl.ds(base, CH1)], ych)
    pltpu.sync_copy(indx.at[pl.ds(base, CH1)], xch)
    z16 = jnp.zeros((16,), _f32)

    def _zb(i, carry):
        zbuf[pl.ds(i * 16, 16)] = z16
        return carry

    lax.fori_loop(0, 512, _zb, 0)
    iot, offs = _lane_consts()

    zslices = [(o * 8192, 8192) for o in range(5)] + [(40960, 5120)]
    for f_loc in range(4):
        f = cid * 4 + f_loc
        for (o, sz) in zslices:
            pltpu.sync_copy(zbuf.at[pl.ds(0, sz)],
                            spm.at[pl.ds(sid * SLC + o, sz)])
        plsc.subcore_barrier()

        def _compact(g, cnt):
            t = tch[pl.ds(g * 16, 16)]
            m = t == f
            pos = g * 16 + iot
            mi = jnp.where(m, _i32(1), _i32(0))
            dst = cnt + plsc.cumsum(mi) - 1
            plsc.store_scatter(posb, [dst], pos, mask=m)
            return cnt + jnp.sum(mi)

        cnt = lax.fori_loop(0, CH1 // 16, _compact, _i32(0))
        nsub = (cnt + (SUB - 1)) >> 6

        def _sub(j, carry):
            off = j * SUB
            for q in range(4):
                lv = (off + q * 16 + iot) < cnt
                pos = posb[pl.ds(off + q * 16, 16)]
                pos = jnp.where(lv, pos, _i32(0))
                yv = plsc.load_gather(ych, [pos])
                xv = plsc.load_gather(xch, [pos])
                sb[pl.ds(q * 16, 16)] = yv * WP + xv
                pidb[pl.ds(q * 16, 16)] = pos + base
            pltpu.async_copy(pat2.at[pidb], valb, semv).wait()
            rem = jnp.minimum(cnt - off, SUB)

            def _pidx(p, carry2):
                sv16 = sb[pl.ds((p >> 4) * 16, 16)]
                sbc = _bcast_lane(sv16, p & 15, iot)
                for k in range(12):
                    idxb[p, pl.ds(k * 16, 16)] = sbc + offs[k]
                pltpu.async_copy(valb.at[p], spm.at[idxb.at[p]],
                                 semsc, add=True)

                @pl.when(p >= 1)
                def _():
                    pltpu.make_async_copy(valb.at[p - 1],
                                          spm.at[idxb.at[p - 1]],
                                          semsc).wait()

                return carry2

            lax.fori_loop(0, rem, _pidx, 0)

            @pl.when(rem >= 1)
            def _():
                pltpu.make_async_copy(valb.at[rem - 1],
                                      spm.at[idxb.at[rem - 1]],
                                      semsc).wait()

            return carry

        lax.fori_loop(0, nsub, _sub, 0)
        plsc.subcore_barrier()
        wslices = [(o * 8192, 8192) for o in range(5)] + [(40960, 5120)]
        for (o, sz) in wslices:
            pltpu.sync_copy(spm.at[pl.ds(sid * SLC + o, sz)],
                            wbuf.at[pl.ds(0, sz)])
            pltpu.sync_copy(wbuf.at[pl.ds(0, sz)],
                            vid.at[pl.ds(f * CHWP + sid * SLC + o, sz)])
        plsc.subcore_barrier()


def _make_scatter():
    mesh = plsc.VectorSubcoreMesh(core_axis_name="c", subcore_axis_name="s")
    return pl.kernel(
        _scatter_body,
        out_type=jax.ShapeDtypeStruct((TCHWP,), _f32),
        mesh=mesh,
        compiler_params=pltpu.CompilerParams(needs_layout_passes=False,
                                             use_tc_tiling_on_sc=False),
        scratch_types=[
            pltpu.VMEM((CH1,), _i32),
            pltpu.VMEM((CH1,), _i32),
            pltpu.VMEM((CH1,), _i32),
            pltpu.VMEM((CH1,), _i32),
            pltpu.VMEM((SUB,), _i32),
            pltpu.VMEM((SUB,), _i32),
            pltpu.VMEM((SUB, CP2), _f32),
            pltpu.VMEM((SUB, CP2), _i32),
            pltpu.VMEM((8192,), _f32),
            pltpu.VMEM((8192,), _f32),
            pltpu.VMEM_SHARED((CHWP,), _f32),
            pltpu.SemaphoreType.DMA,
            pltpu.SemaphoreType.DMA,
        ],
    )


def _conv_body(vref, wref, oref):
    x = vref[0][:, :W]
    xl = jnp.concatenate([x[:, 1:2], x[:, :-1]], axis=1)
    xr = jnp.concatenate([x[:, 1:], x[:, -2:-1]], axis=1)
    acc = None
    for jcol, xc in ((0, xl), (1, x), (2, xr)):
        up = jnp.concatenate([xc[1:2], xc[:-1]], axis=0)
        dn = jnp.concatenate([xc[1:], xc[-2:-1]], axis=0)
        for irow, xx in ((0, up), (1, xc), (2, dn)):
            term = xx * wref[irow, jcol]
            acc = term if acc is None else acc + term
    oref[0, :, :W] = acc


def _conv(vid3, w33):
    return pl.pallas_call(
        _conv_body,
        out_shape=jax.ShapeDtypeStruct((T * C, H, WP), _f32),
        grid=(T * C,),
        in_specs=[
            pl.BlockSpec((1, H, WP), lambda i: (i, 0, 0)),
            pl.BlockSpec((3, 3), lambda i: (0, 0)),
        ],
        out_specs=pl.BlockSpec((1, H, WP), lambda i: (i, 0, 0)),
    )(vid3, w33)


def _gather_body(indt, indy, indx, vcv, out, tch, ych, xch,
                 idxa, idxb, gba, gbb, sema, semb):
    cid = lax.axis_index("c")
    sid = lax.axis_index("s")
    wid = sid * 2 + cid
    base = wid * CH2
    pltpu.sync_copy(indt.at[pl.ds(base, CH2)], tch)
    pltpu.sync_copy(indy.at[pl.ds(base, CH2)], ych)
    pltpu.sync_copy(indx.at[pl.ds(base, CH2)], xch)
    iot, offs = _lane_consts()
    obase = base * CP2 // 128

    def _group(g, ib, gb, sem):
        t = tch[pl.ds(g * 16, 16)]
        y = ych[pl.ds(g * 16, 16)]
        x = xch[pl.ds(g * 16, 16)]
        valid = t >= 0
        sv = t * CHWP + y * WP + x
        sv = jnp.where(valid, sv, ((g * 16 + iot) & 1023) * 192)
        for p in range(16):
            sbc = _bcast_lane(sv, p, iot)
            for k in range(12):
                flat = p * CP2 + k * 16
                ib[flat // 128, pl.ds(flat % 128, 16)] = sbc + offs[k]
        for r in range(NR):
            pltpu.async_copy(vcv.at[ib.at[r]], gb.at[r], sem)

    def _drain(g, ib, gb, sem):
        for r in range(NR):
            pltpu.make_async_copy(vcv.at[ib.at[r]], gb.at[r], sem).wait()
        pltpu.sync_copy(gb, out.at[pl.ds(obase + g * NR, NR)])

    def _pair(gg, carry):
        g0 = gg * 2
        _group(g0, idxa, gba, sema)

        @pl.when(gg >= 1)
        def _():
            _drain(g0 - 1, idxb, gbb, semb)

        _group(g0 + 1, idxb, gbb, semb)
        _drain(g0, idxa, gba, sema)
        return carry

    npairs = jnp.where(wid == NW - 1, _i32(LASTP), _i32(NG2 // 2))
    lax.fori_loop(0, npairs, _pair, 0)
    _drain(npairs * 2 - 1, idxb, gbb, semb)


def _make_gather():
    mesh = plsc.VectorSubcoreMesh(core_axis_name="c", subcore_axis_name="s")
    return pl.kernel(
        _gather_body,
        out_type=jax.ShapeDtypeStruct((OROWS, 128), _f32),
        mesh=mesh,
        compiler_params=pltpu.CompilerParams(needs_layout_passes=False,
                                             use_tc_tiling_on_sc=False),
        scratch_types=[
            pltpu.VMEM((CH2,), _i32),
            pltpu.VMEM((CH2,), _i32),
            pltpu.VMEM((CH2,), _i32),
            pltpu.VMEM((NR, 128), _i32),
            pltpu.VMEM((NR, 128), _i32),
            pltpu.VMEM((NR, 128), _f32),
            pltpu.VMEM((NR, 128), _f32),
            pltpu.SemaphoreType.DMA,
            pltpu.SemaphoreType.DMA,
        ],
    )


def kernel(patches, kernel_2d, inds_t, inds_y, inds_x):
    pat2 = patches.reshape(N, CP2)
    pad = NPAD - N
    tpad = jnp.concatenate([inds_t.astype(_i32), jnp.full((pad,), -1, _i32)])
    ypad = jnp.concatenate([inds_y.astype(_i32), jnp.zeros((pad,), _i32)])
    xpad = jnp.concatenate([inds_x.astype(_i32), jnp.zeros((pad,), _i32)])
    vid = _make_scatter()(tpad, ypad, xpad, pat2)
    vcv = _conv(vid.reshape(T * C, H, WP), kernel_2d.reshape(3, 3))
    y2d = _make_gather()(tpad, ypad, xpad, vcv.reshape(TCHWP))
    return y2d.reshape(1, N, 1, CP2)
